# baseline (speedup 1.0000x reference)
import functools

import jax
import jax.numpy as jnp
from jax import lax
from jax.experimental import pallas as pl
from jax.experimental.pallas import tpu as pltpu
from jax.experimental.pallas import tpu_sc as plsc

BATCH = 16384
DIM = 64
LANES = 16

_info = plsc.get_sparse_core_info()
_NC, _NS = _info.num_cores, _info.num_subcores
_NW = _NC * _NS
_BPW = BATCH // _NW
_HALF = _BPW // 2


def _body(lat_hbm, label_hbm, tab_hbm, out_hbm, idx_v, pidx_v, off_v,
          lat_v, pairs_v, out_v, sem):
    wid = lax.axis_index("s") * _NC + lax.axis_index("c")
    base = wid * _BPW

    pltpu.sync_copy(label_hbm.at[pl.ds(base, _BPW)], idx_v)
    pltpu.sync_copy(lat_hbm.at[:, pl.ds(base, _BPW)], lat_v)

    def prep(g, carry):
        sl = pl.ds(g * LANES, LANES)
        v = idx_v[sl]
        pidx_v[sl] = lax.shift_right_logical(v, 1)
        off_v[sl] = lax.shift_left(lax.bitwise_and(v, 1), 6)
        return carry

    lax.fori_loop(0, _BPW // LANES, prep, 0)

    lane = lax.iota(jnp.int32, LANES)

    for h in range(2):
        hb = h * _HALF
        pltpu.async_copy(
            tab_hbm.at[pidx_v.at[pl.ds(hb, _HALF)]], pairs_v, sem
        ).wait()

        def sel_mul(g, carry):
            sl = pl.ds(hb + g * LANES, LANES)
            i0 = lane + g * LANES
            off = off_v[sl]

            def col(j, c):
                vals = plsc.load_gather(pairs_v, [i0, off + j])
                out_v[j, sl] = vals * lat_v[j, sl]
                return c

            lax.fori_loop(0, DIM, col, 0)
            return carry

        lax.fori_loop(0, _HALF // LANES, sel_mul, 0)

    pltpu.sync_copy(out_v, out_hbm.at[:, pl.ds(base, _BPW)])


@jax.jit
def _run(lat_t, label, tab2):
    mesh = plsc.VectorSubcoreMesh(core_axis_name="c", subcore_axis_name="s")
    kern = functools.partial(
        pl.kernel,
        mesh=mesh,
        out_type=jax.ShapeDtypeStruct((DIM, BATCH), jnp.float32),
        scratch_types=[
            pltpu.VMEM((_BPW,), jnp.int32),
            pltpu.VMEM((_BPW,), jnp.int32),
            pltpu.VMEM((_BPW,), jnp.int32),
            pltpu.VMEM((DIM, _BPW), jnp.float32),
            pltpu.VMEM((_HALF, 2 * DIM), jnp.float32),
            pltpu.VMEM((DIM, _BPW), jnp.float32),
            pltpu.SemaphoreType.DMA,
        ],
        compiler_params=pltpu.CompilerParams(needs_layout_passes=False),
    )(_body)
    return kern(lat_t, label, tab2)


def kernel(latent, label, table):
    tab2 = table.reshape(table.shape[0] // 2, 2 * DIM)
    out_t = _run(latent.T, label.astype(jnp.int32), tab2)
    return out_t.T


# speedup vs baseline: 2.1145x; 2.1145x over previous
import functools

import jax
import jax.numpy as jnp
from jax import lax
from jax.experimental import pallas as pl
from jax.experimental.pallas import tpu as pltpu
from jax.experimental.pallas import tpu_sc as plsc

BATCH = 16384
DIM = 64
LANES = 16

_info = plsc.get_sparse_core_info()
_NC, _NS = _info.num_cores, _info.num_subcores
_NW = _NC * _NS
_BPW = BATCH // _NW
_HALF = _BPW // 2


def _body(lat_hbm, label_hbm, tab_hbm, out_hbm, idx_v, pidx_v, off_v,
          lat_v, pairs_v, out_v, sem):
    wid = lax.axis_index("s") * _NC + lax.axis_index("c")
    base = wid * _BPW

    pltpu.sync_copy(label_hbm.at[pl.ds(base, _BPW)], idx_v)
    pltpu.sync_copy(lat_hbm.at[:, pl.ds(base, _BPW)], lat_v)

    def prep(g, carry):
        sl = pl.ds(g * LANES, LANES)
        v = idx_v[sl]
        pidx_v[sl] = lax.bitwise_or(
            lax.shift_left(lax.shift_right_logical(v, 13), 12),
            lax.bitwise_and(v, 4095),
        )
        off_v[sl] = lax.shift_left(
            lax.bitwise_and(lax.shift_right_logical(v, 12), 1), 6
        )
        return carry

    lax.fori_loop(0, _BPW // LANES, prep, 0)

    lane = lax.iota(jnp.int32, LANES)

    for h in range(2):
        hb = h * _HALF
        pltpu.async_copy(
            tab_hbm.at[pidx_v.at[pl.ds(hb, _HALF)]], pairs_v, sem
        ).wait()

        def sel_mul(g, carry):
            sl = pl.ds(hb + g * LANES, LANES)
            i0 = lane + g * LANES
            off = off_v[sl]

            def col(j, c):
                vals = plsc.load_gather(pairs_v, [i0, off + j])
                out_v[j, sl] = vals * lat_v[j, sl]
                return c

            lax.fori_loop(0, DIM, col, 0)
            return carry

        lax.fori_loop(0, _HALF // LANES, sel_mul, 0)

    pltpu.sync_copy(out_v, out_hbm.at[:, pl.ds(base, _BPW)])


_PACK_BLK = 4096


def _pack_body(tab_t_ref, out_ref):
    x = tab_t_ref[...]
    lo = x[:, :_PACK_BLK].T
    hi = x[:, _PACK_BLK:].T
    out_ref[...] = jnp.concatenate([lo, hi], axis=1)


def _pack(tab_t):
    grid = (tab_t.shape[1] + 2 * _PACK_BLK - 1) // (2 * _PACK_BLK)
    return pl.pallas_call(
        _pack_body,
        grid=(grid,),
        in_specs=[pl.BlockSpec((DIM, 2 * _PACK_BLK), lambda p: (0, p))],
        out_specs=pl.BlockSpec((_PACK_BLK, 2 * DIM), lambda p: (p, 0)),
        out_shape=jax.ShapeDtypeStruct((grid * _PACK_BLK, 2 * DIM), jnp.float32),
    )(tab_t)


@jax.jit
def _run(lat_t, label, tab2):
    mesh = plsc.VectorSubcoreMesh(core_axis_name="c", subcore_axis_name="s")
    kern = functools.partial(
        pl.kernel,
        mesh=mesh,
        out_type=jax.ShapeDtypeStruct((DIM, BATCH), jnp.float32),
        scratch_types=[
            pltpu.VMEM((_BPW,), jnp.int32),
            pltpu.VMEM((_BPW,), jnp.int32),
            pltpu.VMEM((_BPW,), jnp.int32),
            pltpu.VMEM((DIM, _BPW), jnp.float32),
            pltpu.VMEM((_HALF, 2 * DIM), jnp.float32),
            pltpu.VMEM((DIM, _BPW), jnp.float32),
            pltpu.SemaphoreType.DMA,
        ],
        compiler_params=pltpu.CompilerParams(needs_layout_passes=False),
    )(_body)
    return kern(lat_t, label, tab2)


def kernel(latent, label, table):
    tab2 = _pack(table.T)
    out_t = _run(latent.T, label.astype(jnp.int32), tab2)
    return out_t.T


# speedup vs baseline: 2.3630x; 1.1175x over previous
import functools

import jax
import jax.numpy as jnp
from jax import lax
from jax.experimental import pallas as pl
from jax.experimental.pallas import tpu as pltpu
from jax.experimental.pallas import tpu_sc as plsc

BATCH = 16384
DIM = 64
LANES = 16

_info = plsc.get_sparse_core_info()
_NC, _NS = _info.num_cores, _info.num_subcores
_NW = _NC * _NS
_BPW = BATCH // _NW
_HALF = _BPW // 2


def _body(lat_hbm, label_hbm, tab_hbm, out_hbm, idx_v, pidx_v, off_v,
          lat_v, pairs_v, out_v, sem):
    wid = lax.axis_index("s") * _NC + lax.axis_index("c")
    base = wid * _BPW

    pltpu.sync_copy(label_hbm.at[pl.ds(base, _BPW)], idx_v)
    pltpu.sync_copy(lat_hbm.at[:, pl.ds(base, _BPW)], lat_v)

    def prep(g, carry):
        sl = pl.ds(g * LANES, LANES)
        v = idx_v[sl]
        pidx_v[sl] = lax.bitwise_or(
            lax.shift_left(lax.shift_right_logical(v, 13), 12),
            lax.bitwise_and(v, 4095),
        )
        off_v[sl] = lax.shift_left(
            lax.bitwise_and(lax.shift_right_logical(v, 12), 1), 6
        )
        return carry

    lax.fori_loop(0, _BPW // LANES, prep, 0)

    lane = lax.iota(jnp.int32, LANES)

    for h in range(2):
        hb = h * _HALF
        pltpu.async_copy(
            tab_hbm.at[pidx_v.at[pl.ds(hb, _HALF)]], pairs_v, sem
        ).wait()

        def sel_mul(g, carry):
            sl = pl.ds(hb + g * LANES, LANES)
            i0 = lane + g * LANES
            off = off_v[sl]

            def col(j, c):
                vals = plsc.load_gather(pairs_v, [i0, off + j])
                out_v[j, sl] = vals * lat_v[j, sl]
                return c

            lax.fori_loop(0, DIM, col, 0)
            return carry

        lax.fori_loop(0, _HALF // LANES, sel_mul, 0)

    pltpu.sync_copy(out_v, out_hbm.at[:, pl.ds(base, _BPW)])


_PACK_BLK = 8192


def _pack_body(tab_t_ref, out_ref):
    x = tab_t_ref[...]
    lo = x[:, :_PACK_BLK].T
    hi = x[:, _PACK_BLK:].T
    out_ref[...] = jnp.concatenate([lo, hi], axis=1)


def _pack(tab_t):
    grid = (tab_t.shape[1] + 2 * _PACK_BLK - 1) // (2 * _PACK_BLK)
    return pl.pallas_call(
        _pack_body,
        grid=(grid,),
        in_specs=[pl.BlockSpec((DIM, 2 * _PACK_BLK), lambda p: (0, p))],
        out_specs=pl.BlockSpec((_PACK_BLK, 2 * DIM), lambda p: (p, 0)),
        out_shape=jax.ShapeDtypeStruct((grid * _PACK_BLK, 2 * DIM), jnp.float32),
    )(tab_t)


@jax.jit
def _run(lat_t, label, tab2):
    mesh = plsc.VectorSubcoreMesh(core_axis_name="c", subcore_axis_name="s")
    kern = functools.partial(
        pl.kernel,
        mesh=mesh,
        out_type=jax.ShapeDtypeStruct((DIM, BATCH), jnp.float32),
        scratch_types=[
            pltpu.VMEM((_BPW,), jnp.int32),
            pltpu.VMEM((_BPW,), jnp.int32),
            pltpu.VMEM((_BPW,), jnp.int32),
            pltpu.VMEM((DIM, _BPW), jnp.float32),
            pltpu.VMEM((_HALF, 2 * DIM), jnp.float32),
            pltpu.VMEM((DIM, _BPW), jnp.float32),
            pltpu.SemaphoreType.DMA,
        ],
        compiler_params=pltpu.CompilerParams(needs_layout_passes=False),
    )(_body)
    return kern(lat_t, label, tab2)


def kernel(latent, label, table):
    tab2 = _pack(table.T)
    out_t = _run(latent.T, label.astype(jnp.int32), tab2)
    return out_t.T


# speedup vs baseline: 2.4951x; 1.0559x over previous
import functools

import jax
import jax.numpy as jnp
from jax import lax
from jax.experimental import pallas as pl
from jax.experimental.pallas import tpu as pltpu
from jax.experimental.pallas import tpu_sc as plsc

BATCH = 16384
DIM = 64
LANES = 16

_info = plsc.get_sparse_core_info()
_NC, _NS = _info.num_cores, _info.num_subcores
_NW = _NC * _NS
_BPW = BATCH // _NW
_HALF = _BPW // 2


def _body(lat_hbm, label_hbm, tab_hbm, out_hbm, idx_v, pidx_v, off_v,
          lat_v, pairs_v, out_v, sem):
    wid = lax.axis_index("s") * _NC + lax.axis_index("c")
    base = wid * _BPW

    pltpu.sync_copy(label_hbm.at[pl.ds(base, _BPW)], idx_v)
    pltpu.sync_copy(lat_hbm.at[:, pl.ds(base, _BPW)], lat_v)

    def prep(g, carry):
        sl = pl.ds(g * LANES, LANES)
        v = idx_v[sl]
        pidx_v[sl] = lax.bitwise_or(
            lax.shift_left(lax.shift_right_logical(v, 13), 12),
            lax.bitwise_and(v, 4095),
        )
        off_v[sl] = lax.shift_left(
            lax.bitwise_and(lax.shift_right_logical(v, 12), 1), 6
        )
        return carry

    lax.fori_loop(0, _BPW // LANES, prep, 0)

    lane = lax.iota(jnp.int32, LANES)

    for h in range(2):
        hb = h * _HALF
        pltpu.async_copy(
            tab_hbm.at[pidx_v.at[pl.ds(hb, _HALF)]], pairs_v, sem
        ).wait()

        def sel_mul(g, carry):
            sl = pl.ds(hb + g * LANES, LANES)
            i0 = lane + g * LANES
            off = off_v[sl]

            def col(j, c):
                vals = plsc.load_gather(pairs_v, [i0, off + j])
                out_v[j, sl] = vals * lat_v[j, sl]
                return c

            lax.fori_loop(0, DIM, col, 0)
            return carry

        lax.fori_loop(0, _HALF // LANES, sel_mul, 0)

    pltpu.sync_copy(out_v, out_hbm.at[:, pl.ds(base, _BPW)])


_PACK_BLK = 16384


def _pack_body(tab_t_ref, out_ref):
    x = tab_t_ref[...]
    lo = x[:, :_PACK_BLK].T
    hi = x[:, _PACK_BLK:].T
    out_ref[...] = jnp.concatenate([lo, hi], axis=1)


def _pack(tab_t):
    grid = (tab_t.shape[1] + 2 * _PACK_BLK - 1) // (2 * _PACK_BLK)
    return pl.pallas_call(
        _pack_body,
        grid=(grid,),
        in_specs=[pl.BlockSpec((DIM, 2 * _PACK_BLK), lambda p: (0, p))],
        out_specs=pl.BlockSpec((_PACK_BLK, 2 * DIM), lambda p: (p, 0)),
        out_shape=jax.ShapeDtypeStruct((grid * _PACK_BLK, 2 * DIM), jnp.float32),
    )(tab_t)


@jax.jit
def _run(lat_t, label, tab2):
    mesh = plsc.VectorSubcoreMesh(core_axis_name="c", subcore_axis_name="s")
    kern = functools.partial(
        pl.kernel,
        mesh=mesh,
        out_type=jax.ShapeDtypeStruct((DIM, BATCH), jnp.float32),
        scratch_types=[
            pltpu.VMEM((_BPW,), jnp.int32),
            pltpu.VMEM((_BPW,), jnp.int32),
            pltpu.VMEM((_BPW,), jnp.int32),
            pltpu.VMEM((DIM, _BPW), jnp.float32),
            pltpu.VMEM((_HALF, 2 * DIM), jnp.float32),
            pltpu.VMEM((DIM, _BPW), jnp.float32),
            pltpu.SemaphoreType.DMA,
        ],
        compiler_params=pltpu.CompilerParams(needs_layout_passes=False),
    )(_body)
    return kern(lat_t, label, tab2)


def kernel(latent, label, table):
    tab2 = _pack(table.T)
    out_t = _run(latent.T, label.astype(jnp.int32), tab2)
    return out_t.T


# speedup vs baseline: 2.5030x; 1.0032x over previous
import functools

import jax
import jax.numpy as jnp
from jax import lax
from jax.experimental import pallas as pl
from jax.experimental.pallas import tpu as pltpu
from jax.experimental.pallas import tpu_sc as plsc

BATCH = 16384
DIM = 64
LANES = 16

_info = plsc.get_sparse_core_info()
_NC, _NS = _info.num_cores, _info.num_subcores
_NW = _NC * _NS
_BPW = BATCH // _NW
_CHUNK = 128


def _body(lat_hbm, label_hbm, tab_hbm, out_hbm, idx_v, pidx_v, off_v,
          lat_v, pairs_v, out_v, sems):
    wid = lax.axis_index("s") * _NC + lax.axis_index("c")
    base = wid * _BPW

    pltpu.sync_copy(label_hbm.at[pl.ds(base, _BPW)], idx_v)
    pltpu.sync_copy(lat_hbm.at[:, pl.ds(base, _BPW)], lat_v)

    def prep(g, carry):
        sl = pl.ds(g * LANES, LANES)
        v = idx_v[sl]
        pidx_v[sl] = lax.bitwise_or(
            lax.shift_left(lax.shift_right_logical(v, 13), 12),
            lax.bitwise_and(v, 4095),
        )
        off_v[sl] = lax.shift_left(
            lax.bitwise_and(lax.shift_right_logical(v, 12), 1), 6
        )
        return carry

    lax.fori_loop(0, _BPW // LANES, prep, 0)

    lane = lax.iota(jnp.int32, LANES)

    def fire(c):
        pltpu.async_copy(
            tab_hbm.at[pidx_v.at[pl.ds(c * _CHUNK, _CHUNK)]],
            pairs_v.at[c % 2],
            sems.at[c % 2],
        )

    fire(0)
    for c in range(_BPW // _CHUNK):
        if c + 1 < _BPW // _CHUNK:
            fire(c + 1)
        pltpu.make_async_copy(
            tab_hbm.at[pidx_v.at[pl.ds(c * _CHUNK, _CHUNK)]],
            pairs_v.at[c % 2],
            sems.at[c % 2],
        ).wait()
        buf = pairs_v.at[c % 2]

        def sel_mul(g, carry):
            sl = pl.ds(c * _CHUNK + g * LANES, LANES)
            i0 = lane + g * LANES
            off = off_v[sl]

            def col(j, cc):
                vals = plsc.load_gather(buf, [i0, off + j])
                out_v[j, sl] = vals * lat_v[j, sl]
                return cc

            lax.fori_loop(0, DIM, col, 0, unroll=8)
            return carry

        lax.fori_loop(0, _CHUNK // LANES, sel_mul, 0)

    pltpu.sync_copy(out_v, out_hbm.at[:, pl.ds(base, _BPW)])


_PACK_BLK = 16384


def _pack_body(tab_t_ref, out_ref):
    x = tab_t_ref[...]
    lo = x[:, :_PACK_BLK].T
    hi = x[:, _PACK_BLK:].T
    out_ref[...] = jnp.concatenate([lo, hi], axis=1)


def _pack(tab_t):
    grid = (tab_t.shape[1] + 2 * _PACK_BLK - 1) // (2 * _PACK_BLK)
    return pl.pallas_call(
        _pack_body,
        grid=(grid,),
        in_specs=[pl.BlockSpec((DIM, 2 * _PACK_BLK), lambda p: (0, p))],
        out_specs=pl.BlockSpec((_PACK_BLK, 2 * DIM), lambda p: (p, 0)),
        out_shape=jax.ShapeDtypeStruct((grid * _PACK_BLK, 2 * DIM), jnp.float32),
        compiler_params=pltpu.CompilerParams(vmem_limit_bytes=128 * 1024 * 1024),
    )(tab_t)


@jax.jit
def _run(lat_t, label, tab2):
    mesh = plsc.VectorSubcoreMesh(core_axis_name="c", subcore_axis_name="s")
    kern = functools.partial(
        pl.kernel,
        mesh=mesh,
        out_type=jax.ShapeDtypeStruct((DIM, BATCH), jnp.float32),
        scratch_types=[
            pltpu.VMEM((_BPW,), jnp.int32),
            pltpu.VMEM((_BPW,), jnp.int32),
            pltpu.VMEM((_BPW,), jnp.int32),
            pltpu.VMEM((DIM, _BPW), jnp.float32),
            pltpu.VMEM((2, _CHUNK, 2 * DIM), jnp.float32),
            pltpu.VMEM((DIM, _BPW), jnp.float32),
            pltpu.SemaphoreType.DMA((2,)),
        ],
        compiler_params=pltpu.CompilerParams(needs_layout_passes=False),
    )(_body)
    return kern(lat_t, label, tab2)


def kernel(latent, label, table):
    tab2 = _pack(table.T)
    out_t = _run(latent.T, label.astype(jnp.int32), tab2)
    return out_t.T
